# Initial kernel scaffold; baseline (speedup 1.0000x reference)
import jax
import jax.numpy as jnp
from jax.experimental import pallas as pl


def kernel(edge_index, edge_weight, users, user_emb, item_emb):
    raise NotImplementedError("write your pallas kernel here")


# speedup vs baseline: 6.7961x; 6.7961x over previous
import functools

import jax
import jax.numpy as jnp
from jax import lax
from jax.experimental import pallas as pl
from jax.experimental.pallas import tpu as pltpu
from jax.experimental.pallas import tpu_sc as plsc

N_CORES = 2
N_SUB = 16
LANES = 16
DIM = 32
CH = 1024
CHR = CH // 128
NBUF = 4
HALF_P = 50176
ROWS_PER_TILE = HALF_P // N_SUB
OUT_CHUNK = 112
N_OUT_CHUNKS = ROWS_PER_TILE // OUT_CHUNK
_GATHER_DNUMS = lax.GatherDimensionNumbers(
    offset_dims=(), collapsed_slice_dims=(0,), start_index_map=(0,))


def _bcast_lane(vec, lane):
    return lax.gather(vec, jnp.full((LANES, 1), lane, jnp.int32),
                      _GATHER_DNUMS, (1,),
                      mode=lax.GatherScatterMode.PROMISE_IN_BOUNDS)


def _make_layer(n_pad, iters):
    mesh = plsc.VectorSubcoreMesh(core_axis_name="c", subcore_axis_name="s")

    @functools.partial(
        pl.kernel,
        out_type=(
            jax.ShapeDtypeStruct((n_pad, DIM), jnp.float32),
            jax.ShapeDtypeStruct((n_pad, DIM), jnp.float32),
        ),
        mesh=mesh,
        compiler_params=pltpu.CompilerParams(use_tc_tiling_on_sc=False),
        scratch_types=[
            pltpu.VMEM_SHARED((HALF_P, DIM), jnp.float32),
            pltpu.VMEM((CHR, 128), jnp.int32),
            pltpu.VMEM((CHR, 128), jnp.int32),
            pltpu.VMEM((CHR, 128), jnp.float32),
            pltpu.VMEM((CHR, 128), jnp.int32),
            pltpu.VMEM((NBUF, 128, DIM), jnp.float32),
            pltpu.VMEM((OUT_CHUNK, DIM), jnp.float32),
            pltpu.VMEM((OUT_CHUNK, DIM), jnp.float32),
            pltpu.SemaphoreType.DMA,
            pltpu.SemaphoreType.DMA,
        ],
    )
    def layer(emb_hbm, src_hbm, dst_hbm, w_hbm, accin_hbm, zeros_hbm,
              newemb_hbm, accout_hbm,
              accum, srcb, dstb, wb, dlb, rowsb, ob_new, ob_acc, gsem, ssem):
        c = lax.axis_index("c")
        s = lax.axis_index("s")
        lo = c * HALF_P
        pltpu.sync_copy(zeros_hbm,
                        accum.at[pl.ds(s * ROWS_PER_TILE, ROWS_PER_TILE)])
        plsc.subcore_barrier()

        def edge_step(it, carry):
            row0 = (s * iters + it) * CHR
            pltpu.sync_copy(src_hbm.at[pl.ds(row0, CHR)], srcb)
            pltpu.sync_copy(dst_hbm.at[pl.ds(row0, CHR)], dstb)
            pltpu.sync_copy(w_hbm.at[pl.ds(row0, CHR)], wb)
            gcp = {}
            scp = {}
            gcp[0] = pltpu.async_copy(emb_hbm.at[srcb.at[0]], rowsb.at[0],
                                      gsem)
            for b in range(CHR):
                q = b % NBUF
                if b + 1 < CHR:
                    if b + 1 >= NBUF:
                        scp[b + 1 - NBUF].wait()
                    gcp[b + 1] = pltpu.async_copy(
                        emb_hbm.at[srcb.at[b + 1]],
                        rowsb.at[(b + 1) % NBUF], gsem)
                gcp[b].wait()
                rq = rowsb.at[q]

                def group_step(g, carry2, b=b, rq=rq):
                    jj = g * LANES
                    d16 = dstb[b, pl.ds(jj, LANES)]
                    w16 = wb[b, pl.ds(jj, LANES)]
                    own = (d16 >= lo) & (d16 < lo + HALF_P)
                    wmk = jnp.where(own, w16, 0.0)
                    dlb[b, pl.ds(jj, LANES)] = jnp.where(own, d16 - lo, 0)
                    for l in range(LANES):
                        wbe = _bcast_lane(wmk, l)
                        for h in range(DIM // LANES):
                            sl = pl.ds(h * LANES, LANES)
                            rq[jj + l, sl] = rq[jj + l, sl] * wbe
                    return carry2

                lax.fori_loop(0, 128 // LANES, group_step, 0)
                scp[b] = pltpu.async_copy(rq, accum.at[dlb.at[b]], ssem,
                                          add=True)
            for b in range(CHR - NBUF, CHR):
                scp[b].wait()
            return carry

        lax.fori_loop(0, iters, edge_step, 0)
        plsc.subcore_barrier()

        def out_step(k, carry):
            r0 = s * ROWS_PER_TILE + k * OUT_CHUNK
            pltpu.sync_copy(accum.at[pl.ds(r0, OUT_CHUNK)], ob_new)
            pltpu.sync_copy(accin_hbm.at[pl.ds(lo + r0, OUT_CHUNK)], ob_acc)

            def add_step(r, carry2):
                for h in range(DIM // LANES):
                    sl = pl.ds(h * LANES, LANES)
                    ob_acc[r, sl] = ob_acc[r, sl] + ob_new[r, sl]
                return carry2

            lax.fori_loop(0, OUT_CHUNK, add_step, 0)
            pltpu.sync_copy(ob_new, newemb_hbm.at[pl.ds(lo + r0, OUT_CHUNK)])
            pltpu.sync_copy(ob_acc, accout_hbm.at[pl.ds(lo + r0, OUT_CHUNK)])
            return carry

        lax.fori_loop(0, N_OUT_CHUNKS, out_step, 0)

    return layer


def _make_gather(b_total):
    bpw = b_total // (N_CORES * N_SUB)
    mesh = plsc.VectorSubcoreMesh(core_axis_name="c", subcore_axis_name="s")

    @functools.partial(
        pl.kernel,
        out_type=jax.ShapeDtypeStruct((b_total, DIM), jnp.float32),
        mesh=mesh,
        compiler_params=pltpu.CompilerParams(use_tc_tiling_on_sc=False),
        scratch_types=[
            pltpu.VMEM((bpw,), jnp.int32),
            pltpu.VMEM((bpw, DIM), jnp.float32),
            pltpu.SemaphoreType.DMA,
        ],
    )
    def gk(table_hbm, idx_hbm, out_hbm, idx_v, rows_v, sem):
        wid = lax.axis_index("s") * N_CORES + lax.axis_index("c")
        base = wid * bpw
        pltpu.sync_copy(idx_hbm.at[pl.ds(base, bpw)], idx_v)
        pltpu.async_copy(table_hbm.at[idx_v], rows_v, sem).wait()
        pltpu.sync_copy(rows_v, out_hbm.at[pl.ds(base, bpw)])

    return gk


def _matmul(uemb, items, n_items):
    bn = 512
    nu = uemb.shape[0]

    def body(u_ref, it_ref, o_ref):
        acc = lax.dot_general(u_ref[...], it_ref[...],
                              (((1,), (1,)), ((), ())),
                              preferred_element_type=jnp.float32)
        o_ref[...] = jax.nn.sigmoid(acc * (1.0 / 16.0))

    return pl.pallas_call(
        body,
        grid=(pl.cdiv(n_items, bn),),
        in_specs=[pl.BlockSpec((nu, DIM), lambda i: (0, 0)),
                  pl.BlockSpec((bn, DIM), lambda i: (i, 0))],
        out_specs=pl.BlockSpec((nu, bn), lambda i: (0, i)),
        out_shape=jax.ShapeDtypeStruct((nu, n_items), jnp.float32),
    )(uemb, items)


def kernel(edge_index, edge_weight, users, user_emb, item_emb):
    n_users, d = user_emb.shape
    n_items = item_emb.shape[0]
    n_pad = 2 * HALF_P
    mid_pad = HALF_P - n_users
    e = edge_weight.shape[0]
    src = edge_index[0].astype(jnp.int32)
    dst = edge_index[1].astype(jnp.int32)
    src_p = jnp.where(src >= n_users, src + mid_pad, src)
    dst_p = jnp.where(dst >= n_users, dst + mid_pad, dst)
    iters = -(-e // (N_SUB * CH))
    e_pad = N_SUB * CH * iters
    padn = e_pad - e
    srcm = jnp.pad(src_p, (0, padn)).reshape(-1, 128)
    dstm = jnp.pad(dst_p, (0, padn)).reshape(-1, 128)
    wm = jnp.pad(edge_weight, (0, padn)).reshape(-1, 128)
    zeros = jnp.zeros((ROWS_PER_TILE, DIM), jnp.float32)
    emb = jnp.concatenate([
        user_emb,
        jnp.zeros((mid_pad, d), jnp.float32),
        item_emb,
        jnp.zeros((mid_pad, d), jnp.float32),
    ], axis=0)
    acc = emb
    layer = _make_layer(n_pad, iters)
    for _ in range(3):
        emb, acc = layer(emb, srcm, dstm, wm, acc, zeros)
    gk = _make_gather(users.shape[0])
    uemb = gk(acc, users.astype(jnp.int32))
    items = lax.slice(acc, (HALF_P, 0), (HALF_P + n_items, DIM))
    return _matmul(uemb, items, n_items)
